# Initial kernel scaffold; baseline (speedup 1.0000x reference)
import jax
import jax.numpy as jnp
from jax.experimental import pallas as pl


def kernel(input, row_idx, col_idx, enc_W1, enc_b1, enc_W2, enc_b2, enc_W3, enc_b3, dec_W1, dec_b1, dec_W2, dec_b2, dec_W3, dec_b3):
    raise NotImplementedError("write your pallas kernel here")


# speedup vs baseline: 1.1574x; 1.1574x over previous
import functools

import jax
import jax.numpy as jnp
from jax import lax
from jax.experimental import pallas as pl
from jax.experimental.pallas import tpu as pltpu
from jax.experimental.pallas import tpu_sc as plsc

NROWS = 10000
NROWS_P = 10112
NNZ_R = 160000
NNZP = 163840
NCORES = 2
NSUB = 16
NW = NCORES * NSUB
CHUNK = 128
RB = 512


def _sc_mesh():
    return plsc.VectorSubcoreMesh(
        core_axis_name="c", subcore_axis_name="s",
        num_cores=NCORES, num_subcores=NSUB)


@functools.partial(jax.jit, static_argnames=("finp",))
def _sc_scatter(x, idx2, zrows, finp):
    ept = NNZP // NSUB
    nch = ept // CHUNK
    rpt = NROWS_P // NSUB

    @functools.partial(
        pl.kernel, mesh=_sc_mesh(),
        out_type=jax.ShapeDtypeStruct((2, NROWS_P, finp), jnp.float32),
        scratch_types=[
            pltpu.VMEM((CHUNK,), jnp.int32),
            pltpu.VMEM((CHUNK, finp), jnp.float32),
            pltpu.VMEM_SHARED((NROWS_P, finp), jnp.float32),
        ],
        compiler_params=pltpu.CompilerParams(use_tc_tiling_on_sc=False))
    def k(x_hbm, idx_hbm, z_hbm, out_hbm, idx_v, xbuf, table):
        cid = lax.axis_index("c")
        sid = lax.axis_index("s")
        pltpu.sync_copy(z_hbm.at[pl.ds(sid * rpt, rpt)],
                        table.at[pl.ds(sid * rpt, rpt)])
        plsc.subcore_barrier()

        def body(j, carry):
            base = sid * ept + j * CHUNK
            pltpu.sync_copy(idx_hbm.at[cid, pl.ds(base, CHUNK)], idx_v)
            pltpu.sync_copy(x_hbm.at[pl.ds(base, CHUNK)], xbuf)
            pltpu.sync_copy(xbuf, table.at[idx_v], add=True)
            return carry

        lax.fori_loop(0, nch, body, 0)
        plsc.subcore_barrier()
        pltpu.sync_copy(table.at[pl.ds(sid * rpt, rpt)],
                        out_hbm.at[cid, pl.ds(sid * rpt, rpt)])

    return k(x, idx2, zrows)


@functools.partial(jax.jit, static_argnames=("foutp",))
def _sc_gather(tr, tc, idx2, foutp):
    epw = NNZP // NW
    nch = epw // CHUNK

    @functools.partial(
        pl.kernel, mesh=_sc_mesh(),
        out_type=(jax.ShapeDtypeStruct((NNZP, foutp), jnp.float32),
                  jax.ShapeDtypeStruct((NNZP, foutp), jnp.float32)),
        scratch_types=[
            pltpu.VMEM((CHUNK,), jnp.int32),
            pltpu.VMEM((CHUNK,), jnp.int32),
            pltpu.VMEM((CHUNK, foutp), jnp.float32),
            pltpu.VMEM((CHUNK, foutp), jnp.float32),
            pltpu.SemaphoreType.DMA,
        ],
        compiler_params=pltpu.CompilerParams(use_tc_tiling_on_sc=False))
    def k(tr_hbm, tc_hbm, idx_hbm, gr_hbm, gc_hbm, ridx, cidx, rbuf, cbuf, sem):
        wid = lax.axis_index("s") * NCORES + lax.axis_index("c")

        def body(j, carry):
            b = wid * epw + j * CHUNK
            pltpu.sync_copy(idx_hbm.at[0, pl.ds(b, CHUNK)], ridx)
            pltpu.sync_copy(idx_hbm.at[1, pl.ds(b, CHUNK)], cidx)
            pltpu.async_copy(tr_hbm.at[ridx], rbuf, sem).wait()
            pltpu.async_copy(tc_hbm.at[cidx], cbuf, sem).wait()
            pltpu.sync_copy(rbuf, gr_hbm.at[pl.ds(b, CHUNK)])
            pltpu.sync_copy(cbuf, gc_hbm.at[pl.ds(b, CHUNK)])
            return carry

        lax.fori_loop(0, nch, body, 0)

    return k(tr, tc, idx2)


def _tc_tables(S_parts, wrc, wg, bvec, invc8, first, tparts):
    nparts = len(S_parts)
    ntab = len(tparts)

    def body(*refs):
        s_refs = refs[:nparts]
        wrc_ref, wg_ref, b_ref, inv_ref = refs[nparts:nparts + 4]
        tout = refs[nparts + 4:nparts + 4 + 2 * ntab]
        sr = jnp.concatenate([r[0] for r in s_refs], axis=1)
        sc = jnp.concatenate([r[1] for r in s_refs], axis=1)
        if first:
            invr = 1.0 / jnp.maximum(sr[:, 5:6], 1.0)
            invc = 1.0 / jnp.maximum(sc[:, 5:6], 1.0)
            inv_out = refs[nparts + 4 + 2 * ntab]
            inv_out[0] = jnp.broadcast_to(invr, (NROWS_P, 8))
            inv_out[1] = jnp.broadcast_to(invc, (NROWS_P, 8))
        else:
            invr = inv_ref[0][:, 0:1]
            invc = inv_ref[1][:, 0:1]
        gsum = jnp.sum(sr, axis=0, keepdims=True)
        gb = jnp.dot(gsum * (1.0 / NNZ_R), wg_ref[...],
                     preferred_element_type=jnp.float32) + b_ref[...]
        mask = lax.broadcasted_iota(jnp.int32, (NROWS_P, 1), 0) < NROWS
        tr = jnp.dot(sr * invr, wrc_ref[0], preferred_element_type=jnp.float32)
        tr = jnp.where(mask, tr + gb, 0.0)
        tc = jnp.dot(sc * invc, wrc_ref[1], preferred_element_type=jnp.float32)
        tc = jnp.where(mask, tc, 0.0)
        off = 0
        for p, w in enumerate(tparts):
            tout[p][...] = tr[:, off:off + w]
            tout[ntab + p][...] = tc[:, off:off + w]
            off += w

    out_shape = ([jax.ShapeDtypeStruct((NROWS_P, w), jnp.float32)
                  for w in tparts] * 2)
    if first:
        out_shape.append(jax.ShapeDtypeStruct((2, NROWS_P, 8), jnp.float32))
    res = pl.pallas_call(
        body, out_shape=out_shape,
        compiler_params=pltpu.CompilerParams(
            vmem_limit_bytes=100 * 1024 * 1024))(
        *S_parts, wrc, wg, bvec, invc8)
    tr_parts, tc_parts = res[:ntab], res[ntab:2 * ntab]
    if first:
        return tr_parts, tc_parts, res[-1]
    return tr_parts, tc_parts


def _tc_bigmm(x_parts, gr_parts, gc_parts, wx_parts, relu):
    foutp = wx_parts[0].shape[1]
    nparts = len(x_parts)
    ntab = len(gr_parts)

    def body(*refs):
        x_refs = refs[:nparts]
        g_refs = refs[nparts:nparts + 2 * ntab]
        w_refs = refs[nparts + 2 * ntab:nparts + 2 * ntab + nparts]
        o_ref = refs[-1]
        t = jnp.concatenate(
            [g_refs[p][...] + g_refs[ntab + p][...] for p in range(ntab)],
            axis=1)
        for xr, wr in zip(x_refs, w_refs):
            t = t + jnp.dot(xr[...], wr[...],
                            preferred_element_type=jnp.float32)
        if relu:
            t = jnp.maximum(t, 0.01 * t)
        o_ref[...] = t

    in_specs = (
        [pl.BlockSpec((RB, x.shape[1]), lambda i: (i, 0)) for x in x_parts]
        + [pl.BlockSpec((RB, g.shape[1]), lambda i: (i, 0))
           for g in list(gr_parts) + list(gc_parts)]
        + [pl.BlockSpec(w.shape, lambda i: (0, 0)) for w in wx_parts])
    return pl.pallas_call(
        body,
        grid=(NNZP // RB,),
        in_specs=in_specs,
        out_specs=pl.BlockSpec((RB, foutp), lambda i: (i, 0)),
        out_shape=jax.ShapeDtypeStruct((NNZP, foutp), jnp.float32),
    )(*x_parts, *gr_parts, *gc_parts, *wx_parts)


def _prep_w(W, b, F, fparts, foutp):
    out = W.shape[0]

    def padT(A, finp):
        return jnp.pad(A.T, ((0, finp - A.shape[1]), (0, foutp - out)))

    wx_full = W[:, 0:F]
    wx_parts, off = [], 0
    for fp_real, fp_pad in fparts:
        wx_parts.append(padT(wx_full[:, off:off + fp_real], fp_pad))
        off += fp_real
    wrc = jnp.stack([padT(W[:, F:2 * F], sum(p for _, p in fparts)),
                     padT(W[:, 2 * F:3 * F], sum(p for _, p in fparts))])
    wg = padT(W[:, 3 * F:4 * F], sum(p for _, p in fparts))
    bv = jnp.pad(b[None, :], ((0, 0), (0, foutp - out)))
    return wx_parts, wrc, wg, bv


def kernel(input, row_idx, col_idx,
           enc_W1, enc_b1, enc_W2, enc_b2, enc_W3, enc_b3,
           dec_W1, dec_b1, dec_W2, dec_b2, dec_W3, dec_b3):
    pad = NNZP - NNZ_R
    idx2 = jnp.stack([
        jnp.concatenate([row_idx, jnp.full((pad,), NROWS, jnp.int32)]),
        jnp.concatenate([col_idx, jnp.full((pad,), NROWS, jnp.int32)]),
    ])
    x1 = jnp.concatenate([input, jnp.ones((NNZ_R, 1), jnp.float32)], axis=1)
    x1 = jnp.pad(x1, ((0, pad), (0, 16 - 6)))
    zeros = {f: jnp.zeros((NROWS_P, f), jnp.float32) for f in (16, 160, 32)}
    inv_dummy = jnp.ones((2, NROWS_P, 8), jnp.float32)

    def gather_parts(tr_parts, tc_parts):
        gr_parts, gc_parts = [], []
        for trp, tcp in zip(tr_parts, tc_parts):
            gr, gc = _sc_gather(trp, tcp, idx2, trp.shape[1])
            gr_parts.append(gr)
            gc_parts.append(gc)
        return gr_parts, gc_parts

    def layer(x_parts, fparts, foutp, tparts, W, b, F, relu, first, invc8):
        wx_parts, wrc, wg, bv = _prep_w(W, b, F, fparts, foutp)
        S_parts = [_sc_scatter(x, idx2, zeros[x.shape[1]], x.shape[1])
                   for x in x_parts]
        if first:
            tr_p, tc_p, invc8 = _tc_tables(S_parts, wrc, wg, bv, inv_dummy,
                                           True, tparts)
        else:
            tr_p, tc_p = _tc_tables(S_parts, wrc, wg, bv, invc8, False, tparts)
        gr_p, gc_p = gather_parts(tr_p, tc_p)
        y = _tc_bigmm(x_parts, gr_p, gc_p, wx_parts, relu)
        return y, invc8

    T160 = [160]
    h, invc8 = layer([x1], [(5, 16)], 160, T160,
                     enc_W1, enc_b1, 5, True, True, None)
    h, _ = layer([h], [(150, 160)], 160, T160,
                 enc_W2, enc_b2, 150, True, False, invc8)
    enc, _ = layer([h], [(150, 160)], 32, [32],
                   enc_W3, enc_b3, 150, False, False, invc8)

    S = _sc_scatter(enc, idx2, zeros[32], 32)
    eye = jnp.broadcast_to(jnp.eye(32, dtype=jnp.float32), (2, 32, 32))
    tr_p, tc_p = _tc_tables([S], eye, jnp.zeros((32, 32), jnp.float32),
                            jnp.zeros((1, 32), jnp.float32), invc8,
                            False, [32])
    (emb_r,), (emb_c,) = gather_parts(tr_p, tc_p)

    h, _ = layer([emb_r, emb_c], [(32, 32), (32, 32)], 160, T160,
                 dec_W1, dec_b1, 64, True, False, invc8)
    h, _ = layer([h], [(150, 160)], 160, T160,
                 dec_W2, dec_b2, 150, True, False, invc8)
    y, _ = layer([h], [(150, 160)], 16, [16],
                 dec_W3, dec_b3, 150, False, False, invc8)
    return y[:NNZ_R, :5]
